# Initial kernel scaffold; baseline (speedup 1.0000x reference)
import jax
import jax.numpy as jnp
from jax.experimental import pallas as pl


def kernel(l_size, c_size, l_edge_index, c_edge_index, l_emb, c_emb, l2c_W1, l2c_b1, l2c_W2, l2c_b2, c2l_W1, c2l_b1, c2l_W2, c2l_b2, l2l_W1, l2l_b1, l2l_W2, l2l_b2, cg_Wih, cg_Whh, cg_bih, cg_bhh, lg_Wih, lg_Whh, lg_bih, lg_bhh):
    raise NotImplementedError("write your pallas kernel here")


# speedup vs baseline: 2.6088x; 2.6088x over previous
import functools

import jax
import jax.numpy as jnp
from jax import lax
from jax.experimental import pallas as pl
from jax.experimental.pallas import tpu as pltpu
from jax.experimental.pallas import tpu_sc as plsc

D = 128
NC = 2
NS = 16
K = 80
N_ROUNDS = 4


def _dotT(x, w):
    return lax.dot_general(x, w, (((1,), (1,)), ((), ())),
                           preferred_element_type=jnp.float32,
                           precision=lax.Precision.HIGHEST)


def _mlp_body(x_ref, w1_ref, b1_ref, w2_ref, b2_ref, o_ref):
    x = x_ref[...]
    h = jnp.maximum(_dotT(x, w1_ref[...]) + b1_ref[...], 0.0)
    o_ref[...] = _dotT(h, w2_ref[...]) + b2_ref[...]


def _mlp(x, w1, b1, w2, b2, block):
    n = x.shape[0]
    return pl.pallas_call(
        _mlp_body,
        grid=(n // block,),
        in_specs=[
            pl.BlockSpec((block, D), lambda i: (i, 0)),
            pl.BlockSpec((D, D), lambda i: (0, 0)),
            pl.BlockSpec((1, D), lambda i: (0, 0)),
            pl.BlockSpec((D, D), lambda i: (0, 0)),
            pl.BlockSpec((1, D), lambda i: (0, 0)),
        ],
        out_specs=pl.BlockSpec((block, D), lambda i: (i, 0)),
        out_shape=jax.ShapeDtypeStruct((n, D), jnp.float32),
    )(x, w1, b1.reshape(1, D), w2, b2.reshape(1, D))


def _gru_gates(gi, gh, h):
    r = jax.nn.sigmoid(gi[:, :D] + gh[:, :D])
    z = jax.nn.sigmoid(gi[:, D:2 * D] + gh[:, D:2 * D])
    n = jnp.tanh(gi[:, 2 * D:] + r * gh[:, 2 * D:])
    return (1.0 - z) * n + z * h


def _gru1_body(agg_ref, h_ref, wih_ref, whh_ref, bih_ref, bhh_ref, o_ref):
    x = agg_ref[0] + agg_ref[1]
    h = h_ref[...]
    gi = _dotT(x, wih_ref[...]) + bih_ref[...]
    gh = _dotT(h, whh_ref[...]) + bhh_ref[...]
    o_ref[...] = _gru_gates(gi, gh, h)


def _gru1(agg, h, wih, whh, bih, bhh, block):
    n = h.shape[0]
    return pl.pallas_call(
        _gru1_body,
        grid=(n // block,),
        in_specs=[
            pl.BlockSpec((NC, block, D), lambda i: (0, i, 0)),
            pl.BlockSpec((block, D), lambda i: (i, 0)),
            pl.BlockSpec((3 * D, D), lambda i: (0, 0)),
            pl.BlockSpec((3 * D, D), lambda i: (0, 0)),
            pl.BlockSpec((1, 3 * D), lambda i: (0, 0)),
            pl.BlockSpec((1, 3 * D), lambda i: (0, 0)),
        ],
        out_specs=pl.BlockSpec((block, D), lambda i: (i, 0)),
        out_shape=jax.ShapeDtypeStruct((n, D), jnp.float32),
    )(agg, h, wih, whh, bih.reshape(1, 3 * D), bhh.reshape(1, 3 * D))


def _gru2_body(agg_ref, x2_ref, h_ref, wa_ref, wb_ref, whh_ref, bih_ref,
               bhh_ref, o_ref):
    x1 = agg_ref[0] + agg_ref[1]
    h = h_ref[...]
    gi = _dotT(x1, wa_ref[...]) + _dotT(x2_ref[...], wb_ref[...]) + bih_ref[...]
    gh = _dotT(h, whh_ref[...]) + bhh_ref[...]
    o_ref[...] = _gru_gates(gi, gh, h)


def _gru2(agg, x2, h, wih, whh, bih, bhh, block):
    n = h.shape[0]
    return pl.pallas_call(
        _gru2_body,
        grid=(n // block,),
        in_specs=[
            pl.BlockSpec((NC, block, D), lambda i: (0, i, 0)),
            pl.BlockSpec((block, D), lambda i: (i, 0)),
            pl.BlockSpec((block, D), lambda i: (i, 0)),
            pl.BlockSpec((3 * D, D), lambda i: (0, 0)),
            pl.BlockSpec((3 * D, D), lambda i: (0, 0)),
            pl.BlockSpec((3 * D, D), lambda i: (0, 0)),
            pl.BlockSpec((1, 3 * D), lambda i: (0, 0)),
            pl.BlockSpec((1, 3 * D), lambda i: (0, 0)),
        ],
        out_specs=pl.BlockSpec((block, D), lambda i: (i, 0)),
        out_shape=jax.ShapeDtypeStruct((n, D), jnp.float32),
    )(agg, x2, h, wih[:, :D], wih[:, D:], whh,
      bih.reshape(1, 3 * D), bhh.reshape(1, 3 * D))


def _segsum(feat, sidx, didx, n_out):
    e = sidx.shape[0]
    epw = e // (NC * NS)
    steps = epw // K
    stripe = NS * K
    n_pad = ((n_out + stripe - 1) // stripe) * stripe
    rps = n_pad // NS
    mesh = plsc.VectorSubcoreMesh(core_axis_name="c", subcore_axis_name="s")

    @functools.partial(
        pl.kernel,
        mesh=mesh,
        out_type=jax.ShapeDtypeStruct((NC * n_pad, D), jnp.float32),
        scratch_types=[
            pltpu.VMEM((K,), jnp.int32),
            pltpu.VMEM((K,), jnp.int32),
            pltpu.VMEM((K, D), jnp.float32),
            pltpu.VMEM_SHARED((n_pad, D), jnp.float32),
            pltpu.SemaphoreType.DMA,
        ],
    )
    def seg(feat_hbm, sidx_hbm, didx_hbm, zeros_hbm, out_hbm,
            sidx_v, didx_v, rows_v, acc_sh, sem):
        cid = lax.axis_index("c")
        sid = lax.axis_index("s")
        pltpu.sync_copy(zeros_hbm, acc_sh.at[pl.ds(sid * rps, rps)])
        plsc.subcore_barrier()
        wid = cid * NS + sid
        base0 = wid * epw

        def step(i, carry):
            b = base0 + i * K
            pltpu.sync_copy(sidx_hbm.at[pl.ds(b, K)], sidx_v)
            pltpu.sync_copy(didx_hbm.at[pl.ds(b, K)], didx_v)
            pltpu.async_copy(feat_hbm.at[sidx_v], rows_v, sem).wait()
            pltpu.sync_copy(rows_v, acc_sh.at[didx_v], add=True)
            return carry

        lax.fori_loop(0, steps, step, 0)
        plsc.subcore_barrier()
        pltpu.sync_copy(acc_sh.at[pl.ds(sid * rps, rps)],
                        out_hbm.at[pl.ds(cid * n_pad + sid * rps, rps)])

    zeros = jnp.zeros((rps, D), jnp.float32)
    out = seg(feat, sidx, didx, zeros)
    return out.reshape(NC, n_pad, D)


def kernel(l_size, c_size, l_edge_index, c_edge_index, l_emb, c_emb,
           l2c_W1, l2c_b1, l2c_W2, l2c_b2,
           c2l_W1, c2l_b1, c2l_W2, c2l_b2,
           l2l_W1, l2l_b1, l2l_W2, l2l_b2,
           cg_Wih, cg_Whh, cg_bih, cg_bhh,
           lg_Wih, lg_Whh, lg_bih, lg_bhh):
    n_l = l_emb.shape[0]
    n_c = c_emb.shape[0]
    li = l_edge_index.astype(jnp.int32)
    ci = c_edge_index.astype(jnp.int32)
    bl = 1000
    bc = 1000
    l_embs = [l_emb]
    c_embs = [c_emb]
    for _ in range(N_ROUNDS):
        l_msg = _mlp(l_emb, l2c_W1, l2c_b1, l2c_W2, l2c_b2, bl)
        c_msg = _mlp(c_emb, c2l_W1, c2l_b1, c2l_W2, c2l_b2, bc)
        l2l_in = l_emb.reshape(-1, 2, D)[:, ::-1, :].reshape(-1, D)
        l2l_msg = _mlp(l2l_in, l2l_W1, l2l_b1, l2l_W2, l2l_b2, bl)
        l2c_p = _segsum(l_msg, li, ci, n_c)
        c2l_p = _segsum(c_msg, ci, li, n_l)
        c_emb = _gru1(l2c_p[:, :n_c], c_emb, cg_Wih, cg_Whh, cg_bih, cg_bhh, bc)
        l_emb = _gru2(c2l_p[:, :n_l], l2l_msg, l_emb,
                      lg_Wih, lg_Whh, lg_bih, lg_bhh, bl)
        l_embs.append(l_emb)
        c_embs.append(c_emb)
    return jnp.stack(l_embs), jnp.stack(c_embs)
